# Initial kernel scaffold; baseline (speedup 1.0000x reference)
import jax
import jax.numpy as jnp
from jax.experimental import pallas as pl


def kernel(x, lora_A, lora_B, prototypes, scaling):
    raise NotImplementedError("write your pallas kernel here")


# speedup vs baseline: 2.0466x; 2.0466x over previous
import functools

import jax
import jax.numpy as jnp
from jax.experimental import pallas as pl

_EPS = 1e-8


def _fused_block(x_ref, p_ref, a_ref, b_ref, o_ref, *, rank):
    xb = x_ref[:, :]
    p = p_ref[:, :]
    tn = xb.shape[0]
    e = p.shape[0]
    er = a_ref.shape[0]

    xnorm = jnp.sqrt(jnp.sum(xb * xb, axis=1, keepdims=True))
    pnorm = jnp.sqrt(jnp.sum(p * p, axis=1, keepdims=True))
    xn = xb / (xnorm + _EPS)
    pn = p / (pnorm + _EPS)
    s = jax.lax.dot_general(
        xn, pn, (((1,), (1,)), ((), ())),
        preferred_element_type=jnp.float32)
    sim = jnp.abs(s)

    iota_e = jax.lax.broadcasted_iota(jnp.int32, (tn, e), 1)
    m1 = jnp.max(sim, axis=1, keepdims=True)
    idx1 = jnp.min(jnp.where(sim == m1, iota_e, e), axis=1, keepdims=True)
    masked = jnp.where(iota_e == idx1, -1.0, sim)
    m2 = jnp.max(masked, axis=1, keepdims=True)
    idx2 = jnp.min(jnp.where(masked == m2, iota_e, e), axis=1, keepdims=True)
    c1 = jax.nn.sigmoid(m1 - m2)
    c2 = jax.nn.sigmoid(m2 - m1)

    col_e = jax.lax.broadcasted_iota(jnp.int32, (1, er), 1) // rank
    w = (jnp.where(col_e == idx1, c1, 0.0)
         + jnp.where(col_e == idx2, c2, 0.0))

    z = jax.lax.dot_general(
        xb.astype(jnp.bfloat16), a_ref[:, :], (((1,), (1,)), ((), ())),
        preferred_element_type=jnp.float32)
    u = (z * w).astype(jnp.bfloat16)
    delta = jax.lax.dot_general(
        u, b_ref[:, :], (((1,), (0,)), ((), ())),
        preferred_element_type=jnp.float32)
    o_ref[:, :] = delta


def kernel(x, lora_A, lora_B, prototypes, scaling):
    bsz, seq, d = x.shape
    e, r, _ = lora_A.shape
    n = bsz * seq
    flat_x = x.reshape(n, d)
    a_stack = lora_A.reshape(e * r, d).astype(jnp.bfloat16)
    b_stack = (lora_B.transpose(0, 2, 1).reshape(e * r, d)
               * jnp.float32(scaling)).astype(jnp.bfloat16)

    tn = 256
    grid = (n // tn,)
    out = pl.pallas_call(
        functools.partial(_fused_block, rank=r),
        grid=grid,
        in_specs=[
            pl.BlockSpec((tn, d), lambda i: (i, 0)),
            pl.BlockSpec((e, d), lambda i: (0, 0)),
            pl.BlockSpec((e * r, d), lambda i: (0, 0)),
            pl.BlockSpec((e * r, d), lambda i: (0, 0)),
        ],
        out_specs=pl.BlockSpec((tn, d), lambda i: (i, 0)),
        out_shape=jax.ShapeDtypeStruct((n, d), jnp.float32),
    )(flat_x, prototypes, a_stack, b_stack)
    return out.reshape(bsz, seq, d)


# speedup vs baseline: 2.4723x; 1.2080x over previous
import functools

import jax
import jax.numpy as jnp
from jax.experimental import pallas as pl

_EPS = 1e-8


def _fused_block(x_ref, p_ref, a_ref, b_ref, o_ref, *, rank):
    xb = x_ref[:, :]
    p = p_ref[:, :]
    tn = xb.shape[0]
    e = p.shape[0]
    er = a_ref.shape[0]

    xnorm = jnp.sqrt(jnp.sum(xb * xb, axis=1, keepdims=True))
    pnorm = jnp.sqrt(jnp.sum(p * p, axis=1, keepdims=True))
    xn = xb / (xnorm + _EPS)
    pn = p / (pnorm + _EPS)
    s = jax.lax.dot_general(
        xn, pn, (((1,), (1,)), ((), ())),
        preferred_element_type=jnp.float32)
    sim = jnp.abs(s)

    iota_e = jax.lax.broadcasted_iota(jnp.int32, (tn, e), 1)
    m1 = jnp.max(sim, axis=1, keepdims=True)
    idx1 = jnp.min(jnp.where(sim == m1, iota_e, e), axis=1, keepdims=True)
    masked = jnp.where(iota_e == idx1, -1.0, sim)
    m2 = jnp.max(masked, axis=1, keepdims=True)
    idx2 = jnp.min(jnp.where(masked == m2, iota_e, e), axis=1, keepdims=True)
    c1 = jax.nn.sigmoid(m1 - m2)
    c2 = jax.nn.sigmoid(m2 - m1)

    col_e = jax.lax.broadcasted_iota(jnp.int32, (1, er), 1) // rank
    w = (jnp.where(col_e == idx1, c1, 0.0)
         + jnp.where(col_e == idx2, c2, 0.0))

    z = jax.lax.dot_general(
        xb.astype(jnp.bfloat16), a_ref[:, :], (((1,), (1,)), ((), ())),
        preferred_element_type=jnp.float32)
    u = (z * w).astype(jnp.bfloat16)
    delta = jax.lax.dot_general(
        u, b_ref[:, :], (((1,), (0,)), ((), ())),
        preferred_element_type=jnp.float32)
    o_ref[:, :] = delta


def kernel(x, lora_A, lora_B, prototypes, scaling):
    bsz, seq, d = x.shape
    e, r, _ = lora_A.shape
    n = bsz * seq
    flat_x = x.reshape(n, d)
    a_stack = lora_A.reshape(e * r, d).astype(jnp.bfloat16)
    b_stack = (lora_B.transpose(0, 2, 1).reshape(e * r, d)
               * jnp.float32(scaling)).astype(jnp.bfloat16)

    tn = 512
    grid = (n // tn,)
    out = pl.pallas_call(
        functools.partial(_fused_block, rank=r),
        grid=grid,
        in_specs=[
            pl.BlockSpec((tn, d), lambda i: (i, 0)),
            pl.BlockSpec((e, d), lambda i: (0, 0)),
            pl.BlockSpec((e * r, d), lambda i: (0, 0)),
            pl.BlockSpec((e * r, d), lambda i: (0, 0)),
        ],
        out_specs=pl.BlockSpec((tn, d), lambda i: (i, 0)),
        out_shape=jax.ShapeDtypeStruct((n, d), jnp.float32),
    )(flat_x, prototypes, a_stack, b_stack)
    return out.reshape(bsz, seq, d)


# speedup vs baseline: 2.5777x; 1.0426x over previous
import functools

import jax
import jax.numpy as jnp
from jax.experimental import pallas as pl

_EPS = 1e-8


def _fused_block(x_ref, p_ref, a_ref, b_ref, o_ref, *, rank):
    xb = x_ref[:, :]
    p = p_ref[:, :]
    tn = xb.shape[0]
    e = p.shape[0]
    er = a_ref.shape[0]

    xnorm = jnp.sqrt(jnp.sum(xb * xb, axis=1, keepdims=True))
    pnorm = jnp.sqrt(jnp.sum(p * p, axis=1, keepdims=True))
    xn = xb / (xnorm + _EPS)
    pn = p / (pnorm + _EPS)
    s = jax.lax.dot_general(
        xn, pn, (((1,), (1,)), ((), ())),
        preferred_element_type=jnp.float32)
    sim = jnp.abs(s)

    iota_e = jax.lax.broadcasted_iota(jnp.int32, (tn, e), 1)
    m1 = jnp.max(sim, axis=1, keepdims=True)
    idx1 = jnp.min(jnp.where(sim == m1, iota_e, e), axis=1, keepdims=True)
    masked = jnp.where(iota_e == idx1, -1.0, sim)
    m2 = jnp.max(masked, axis=1, keepdims=True)
    idx2 = jnp.min(jnp.where(masked == m2, iota_e, e), axis=1, keepdims=True)
    c1 = jax.nn.sigmoid(m1 - m2)
    c2 = jax.nn.sigmoid(m2 - m1)

    col_e = jax.lax.broadcasted_iota(jnp.int32, (1, er), 1) // rank
    w = (jnp.where(col_e == idx1, c1, 0.0)
         + jnp.where(col_e == idx2, c2, 0.0))

    z = jax.lax.dot_general(
        xb.astype(jnp.bfloat16), a_ref[:, :], (((1,), (1,)), ((), ())),
        preferred_element_type=jnp.float32)
    u = (z * w).astype(jnp.bfloat16)
    delta = jax.lax.dot_general(
        u, b_ref[:, :], (((1,), (0,)), ((), ())),
        preferred_element_type=jnp.float32)
    o_ref[:, :] = delta


def kernel(x, lora_A, lora_B, prototypes, scaling):
    bsz, seq, d = x.shape
    e, r, _ = lora_A.shape
    n = bsz * seq
    flat_x = x.reshape(n, d)
    a_stack = lora_A.reshape(e * r, d).astype(jnp.bfloat16)
    b_stack = (lora_B.transpose(0, 2, 1).reshape(e * r, d)
               * jnp.float32(scaling)).astype(jnp.bfloat16)

    tn = 1024
    grid = (n // tn,)
    out = pl.pallas_call(
        functools.partial(_fused_block, rank=r),
        grid=grid,
        in_specs=[
            pl.BlockSpec((tn, d), lambda i: (i, 0)),
            pl.BlockSpec((e, d), lambda i: (0, 0)),
            pl.BlockSpec((e * r, d), lambda i: (0, 0)),
            pl.BlockSpec((e * r, d), lambda i: (0, 0)),
        ],
        out_specs=pl.BlockSpec((tn, d), lambda i: (i, 0)),
        out_shape=jax.ShapeDtypeStruct((n, d), jnp.float32),
    )(flat_x, prototypes, a_stack, b_stack)
    return out.reshape(bsz, seq, d)
